# initial kernel scaffold (unmeasured)
import jax
import jax.numpy as jnp
from jax import lax
from jax.experimental import pallas as pl
from jax.experimental.pallas import tpu as pltpu


def kernel(
    x,
):
    def body(*refs):
        pass

    out_shape = jax.ShapeDtypeStruct(..., jnp.float32)
    return pl.pallas_call(body, out_shape=out_shape)(...)


# baseline (device time: 30289 ns/iter reference)
import jax
import jax.numpy as jnp
from jax import lax
from jax.experimental import pallas as pl
from jax.experimental.pallas import tpu as pltpu


def kernel(x):
    m_per, n = x.shape

    def body(x_ref, out_ref, comm_ref, send_sem, recv_sem):
        my_x = lax.axis_index("x")
        my_y = lax.axis_index("y")
        my_z = lax.axis_index("z")
        peer = (1 - my_x, my_y, my_z)

        comm_ref[...] = x_ref[...].astype(jnp.bfloat16)

        barrier_sem = pltpu.get_barrier_semaphore()
        pl.semaphore_signal(
            barrier_sem, inc=1, device_id=peer,
            device_id_type=pl.DeviceIdType.MESH,
        )
        pl.semaphore_wait(barrier_sem, 1)

        rdma = pltpu.make_async_remote_copy(
            src_ref=comm_ref,
            dst_ref=out_ref.at[pl.ds(my_x * m_per, m_per), :],
            send_sem=send_sem,
            recv_sem=recv_sem,
            device_id=peer,
            device_id_type=pl.DeviceIdType.MESH,
        )
        rdma.start()
        out_ref[pl.ds(my_x * m_per, m_per), :] = comm_ref[...]
        rdma.wait()

    return pl.pallas_call(
        body,
        out_shape=jax.ShapeDtypeStruct((2 * m_per, n), jnp.bfloat16),
        in_specs=[pl.BlockSpec(memory_space=pltpu.VMEM)],
        out_specs=pl.BlockSpec(memory_space=pltpu.VMEM),
        scratch_shapes=[
            pltpu.VMEM((m_per, n), jnp.bfloat16),
            pltpu.SemaphoreType.DMA,
            pltpu.SemaphoreType.DMA,
        ],
        compiler_params=pltpu.CompilerParams(collective_id=0),
    )(x)


# device time: 22652 ns/iter; 1.3371x vs baseline; 1.3371x over previous
import jax
import jax.numpy as jnp
from jax import lax
from jax.experimental import pallas as pl
from jax.experimental.pallas import tpu as pltpu

K = 16


def kernel(x):
    m_per, n = x.shape
    half = m_per // 2
    ch = half // K

    def body(x_ref, out_ref, stage_ref, x_send, x_recv, y_send, y_recv):
        my_x = lax.axis_index("x")
        my_y = lax.axis_index("y")
        my_z = lax.axis_index("z")
        xpeer = (1 - my_x, my_y, my_z)
        ypeer = (my_x, 1 - my_y, my_z)

        stage_ref[...] = x_ref[pl.ds(my_y * half, half), :].astype(jnp.bfloat16)

        barrier_sem = pltpu.get_barrier_semaphore()
        for nbr in (xpeer, ypeer):
            pl.semaphore_signal(
                barrier_sem, inc=1, device_id=nbr,
                device_id_type=pl.DeviceIdType.MESH,
            )
        pl.semaphore_wait(barrier_sem, 2)

        x_base = my_x * m_per + my_y * half
        r_base = (1 - my_x) * m_per + my_y * half

        x_rdmas = []
        for k in range(K):
            r = pltpu.make_async_remote_copy(
                src_ref=stage_ref.at[pl.ds(k * ch, ch), :],
                dst_ref=out_ref.at[pl.ds(x_base + k * ch, ch), :],
                send_sem=x_send.at[k],
                recv_sem=x_recv.at[k],
                device_id=xpeer,
                device_id_type=pl.DeviceIdType.MESH,
            )
            r.start()
            x_rdmas.append(r)

        out_ref[pl.ds(my_x * m_per, m_per), :] = x_ref[...].astype(jnp.bfloat16)

        y_rdmas = []
        for k in range(K):
            x_rdmas[k].wait_recv()
            r = pltpu.make_async_remote_copy(
                src_ref=out_ref.at[pl.ds(r_base + k * ch, ch), :],
                dst_ref=out_ref.at[pl.ds(r_base + k * ch, ch), :],
                send_sem=y_send.at[k],
                recv_sem=y_recv.at[k],
                device_id=ypeer,
                device_id_type=pl.DeviceIdType.MESH,
            )
            r.start()
            y_rdmas.append(r)

        for k in range(K):
            x_rdmas[k].wait_send()
            y_rdmas[k].wait_send()
            y_rdmas[k].wait_recv()

    return pl.pallas_call(
        body,
        out_shape=jax.ShapeDtypeStruct((2 * m_per, n), jnp.bfloat16),
        in_specs=[pl.BlockSpec(memory_space=pltpu.VMEM)],
        out_specs=pl.BlockSpec(memory_space=pltpu.VMEM),
        scratch_shapes=[
            pltpu.VMEM((half, n), jnp.bfloat16),
            pltpu.SemaphoreType.DMA((K,)),
            pltpu.SemaphoreType.DMA((K,)),
            pltpu.SemaphoreType.DMA((K,)),
            pltpu.SemaphoreType.DMA((K,)),
        ],
        compiler_params=pltpu.CompilerParams(collective_id=0),
    )(x)


# device time: 20811 ns/iter; 1.4554x vs baseline; 1.0885x over previous
import os

import jax
import jax.numpy as jnp
from jax import lax
from jax.experimental import pallas as pl
from jax.experimental.pallas import tpu as pltpu

K = 16


def kernel(x):
    m_per, n = x.shape
    half = m_per // 2
    ch = half // K

    def body(x_ref, out_ref, stage_ref, x_send, x_recv, y_send, y_recv):
        my_x = lax.axis_index("x")
        my_y = lax.axis_index("y")
        my_z = lax.axis_index("z")
        xpeer = (1 - my_x, my_y, my_z)
        ypeer = (my_x, 1 - my_y, my_z)

        stage_ref[...] = x_ref[pl.ds(my_y * half, half), :].astype(jnp.bfloat16)

        barrier_sem = pltpu.get_barrier_semaphore()
        for nbr in (xpeer, ypeer):
            pl.semaphore_signal(
                barrier_sem, inc=1, device_id=nbr,
                device_id_type=pl.DeviceIdType.MESH,
            )
        pl.semaphore_wait(barrier_sem, 2)

        x_base = my_x * m_per + my_y * half
        r_base = (1 - my_x) * m_per + my_y * half

        x_rdmas = []
        for k in range(K):
            r = pltpu.make_async_remote_copy(
                src_ref=stage_ref.at[pl.ds(k * ch, ch), :],
                dst_ref=out_ref.at[pl.ds(x_base + k * ch, ch), :],
                send_sem=x_send.at[k],
                recv_sem=x_recv.at[k],
                device_id=xpeer,
                device_id_type=pl.DeviceIdType.MESH,
            )
            r.start()
            x_rdmas.append(r)

        out_ref[pl.ds(my_x * m_per, m_per), :] = x_ref[...].astype(jnp.bfloat16)

        y_rdmas = []
        for k in range(K):
            x_rdmas[k].wait_recv()
            r = pltpu.make_async_remote_copy(
                src_ref=out_ref.at[pl.ds(r_base + k * ch, ch), :],
                dst_ref=out_ref.at[pl.ds(r_base + k * ch, ch), :],
                send_sem=y_send.at[k],
                recv_sem=y_recv.at[k],
                device_id=ypeer,
                device_id_type=pl.DeviceIdType.MESH,
            )
            r.start()
            y_rdmas.append(r)

        for k in range(K):
            x_rdmas[k].wait_send()
            y_rdmas[k].wait_send()
            y_rdmas[k].wait_recv()

    return pl.pallas_call(
        body,
        out_shape=jax.ShapeDtypeStruct((2 * m_per, n), jnp.bfloat16),
        in_specs=[pl.BlockSpec(memory_space=pltpu.VMEM)],
        out_specs=pl.BlockSpec(memory_space=pltpu.VMEM),
        scratch_shapes=[
            pltpu.VMEM((half, n), jnp.bfloat16),
            pltpu.SemaphoreType.DMA((K,)),
            pltpu.SemaphoreType.DMA((K,)),
            pltpu.SemaphoreType.DMA((K,)),
            pltpu.SemaphoreType.DMA((K,)),
        ],
        compiler_params=pltpu.CompilerParams(collective_id=0),
    )(x)


def _make_bench_kernel(variant):

    def kern(x):
        m_per, n = x.shape
        half = m_per // 2
        ch = half // K

        def body(x_ref, out_ref, stage_ref, s_send, s_recv, t_send, t_recv):
            my_x = lax.axis_index("x")
            my_y = lax.axis_index("y")
            my_z = lax.axis_index("z")
            xpeer = (1 - my_x, my_y, my_z)
            ypeer = (my_x, 1 - my_y, my_z)

            stage_ref[...] = x_ref[:half, :].astype(jnp.bfloat16)

            nbrs = [xpeer] if variant == "xonly" else [xpeer, ypeer]
            barrier_sem = pltpu.get_barrier_semaphore()
            for nbr in nbrs:
                pl.semaphore_signal(
                    barrier_sem, inc=1, device_id=nbr,
                    device_id_type=pl.DeviceIdType.MESH,
                )
            pl.semaphore_wait(barrier_sem, len(nbrs))

            rdmas = []
            for k in range(K):
                r = pltpu.make_async_remote_copy(
                    src_ref=stage_ref.at[pl.ds(k * ch, ch), :],
                    dst_ref=out_ref.at[pl.ds(k * ch, ch), :],
                    send_sem=s_send.at[k], recv_sem=s_recv.at[k],
                    device_id=xpeer, device_id_type=pl.DeviceIdType.MESH,
                )
                r.start()
                rdmas.append(r)
                if variant == "xy_indep":
                    r2 = pltpu.make_async_remote_copy(
                        src_ref=stage_ref.at[pl.ds(k * ch, ch), :],
                        dst_ref=out_ref.at[pl.ds(half + k * ch, ch), :],
                        send_sem=t_send.at[k], recv_sem=t_recv.at[k],
                        device_id=ypeer, device_id_type=pl.DeviceIdType.MESH,
                    )
                    r2.start()
                    rdmas.append(r2)

            for r in rdmas:
                r.wait_send()
            for r in rdmas:
                r.wait_recv()

        return pl.pallas_call(
            body,
            out_shape=jax.ShapeDtypeStruct((2 * m_per, n), jnp.bfloat16),
            in_specs=[pl.BlockSpec(memory_space=pltpu.VMEM)],
            out_specs=pl.BlockSpec(memory_space=pltpu.VMEM),
            scratch_shapes=[
                pltpu.VMEM((half, n), jnp.bfloat16),
                pltpu.SemaphoreType.DMA((K,)),
                pltpu.SemaphoreType.DMA((K,)),
                pltpu.SemaphoreType.DMA((K,)),
                pltpu.SemaphoreType.DMA((K,)),
            ],
            compiler_params=pltpu.CompilerParams(collective_id=0),
        )(x)

    return kern


_BENCH = os.environ.get("BENCH_VARIANT")
if _BENCH:
    kernel = _make_bench_kernel(_BENCH)
